# baseline (speedup 1.0000x reference)
import functools

import jax
import jax.numpy as jnp
from jax import lax
from jax.experimental import pallas as pl
from jax.experimental.pallas import tpu as pltpu
from jax.experimental.pallas import tpu_sc as plsc

N, D, E, B, O = 10000, 128, 320000, 16, 10

NC, NS = 2, 16
NW = NC * NS
EPW = E // NW
CHUNK = 32
NCHUNK = EPW // CHUNK
TAIL = EPW - NCHUNK * CHUNK
NP = 10240
RPT = NP // NS

R = 1000
NB = N // R


NR = 8
NG = NCHUNK // NR


def _sc_agg_body(src_hbm, dst_hbm, feat_hbm, zeros_hbm, out_hbm,
                 sidx_f, didx_r, didx_t, rows_v, rows_t, acc_sh, *sems):
    gsems = sems[:NR]
    dsems = sems[NR:]
    c = lax.axis_index("c")
    s = lax.axis_index("s")
    wid = c * NS + s
    @pl.when(c == 0)
    def _():
        @pl.when(s < NS - 1)
        def _():
            pltpu.sync_copy(feat_hbm.at[pl.ds(s * RPT, RPT)],
                            acc_sh.at[pl.ds(s * RPT, RPT)])

        @pl.when(s == NS - 1)
        def _():
            lo = (NS - 1) * RPT
            pltpu.sync_copy(feat_hbm.at[pl.ds(lo, N - lo)],
                            acc_sh.at[pl.ds(lo, N - lo)])

    @pl.when(c == 1)
    def _():
        pltpu.sync_copy(zeros_hbm.at[pl.ds(s * RPT, RPT)],
                        acc_sh.at[pl.ds(s * RPT, RPT)])
    ebase = wid * EPW
    pltpu.sync_copy(src_hbm.at[pl.ds(ebase, EPW)], sidx_f)
    plsc.subcore_barrier()

    def gather(chunk, slot):
        return pltpu.make_async_copy(
            feat_hbm.at[sidx_f.at[pl.ds(chunk * CHUNK, CHUNK)]],
            rows_v.at[slot], gsems[slot])

    def dfetch(chunk, slot):
        return pltpu.make_async_copy(
            dst_hbm.at[pl.ds(ebase + chunk * CHUNK, CHUNK)],
            didx_r.at[slot], dsems[slot])

    for b in range(NR):
        dfetch(b, b).start()
        gather(b, b).start()

    def round_(g, carry):
        for b in range(NR):
            ck = g * NR + b
            gather(ck, b).wait()
            dfetch(ck, b).wait()
            pltpu.sync_copy(rows_v.at[b], acc_sh.at[didx_r.at[b]], add=True)

            @pl.when(g < NG - 1)
            def _():
                dfetch(ck + NR, b).start()
                gather(ck + NR, b).start()
        return carry

    lax.fori_loop(0, NG, round_, 0)
    pltpu.sync_copy(dst_hbm.at[pl.ds(ebase + NCHUNK * CHUNK, TAIL)], didx_t)
    pltpu.async_copy(
        feat_hbm.at[sidx_f.at[pl.ds(NCHUNK * CHUNK, TAIL)]], rows_t,
        gsems[0]).wait()
    pltpu.sync_copy(rows_t, acc_sh.at[didx_t], add=True)
    plsc.subcore_barrier()
    pltpu.sync_copy(acc_sh.at[pl.ds(s * RPT, RPT)],
                    out_hbm.at[c, pl.ds(s * RPT, RPT)])


_sc_agg = functools.partial(
    pl.kernel,
    out_type=jax.ShapeDtypeStruct((NC, NP, D), jnp.float32),
    mesh=plsc.VectorSubcoreMesh(core_axis_name="c", subcore_axis_name="s"),
    scratch_types=[
        pltpu.VMEM((EPW,), jnp.int32),
        pltpu.VMEM((NR, CHUNK), jnp.int32),
        pltpu.VMEM((TAIL,), jnp.int32),
        pltpu.VMEM((NR, CHUNK, D), jnp.float32),
        pltpu.VMEM((TAIL, D), jnp.float32),
        pltpu.VMEM_SHARED((NP, D), jnp.float32),
    ] + [pltpu.SemaphoreType.DMA] * (2 * NR),
)(_sc_agg_body)


def _gin_mlp(p_ref, w1_ref, b1_ref, w2_ref, b2_ref):
    h = p_ref[0] + p_ref[1]
    a = jnp.maximum(
        jnp.dot(h, w1_ref[...], preferred_element_type=jnp.float32)
        + b1_ref[...], 0.0)
    return (jnp.dot(a, w2_ref[...], preferred_element_type=jnp.float32)
            + b2_ref[...])


def _bn_stats(stats, h2, i):
    s1 = jnp.sum(h2, axis=0, keepdims=True)
    s2 = jnp.sum(h2 * h2, axis=0, keepdims=True)

    @pl.when(i == 0)
    def _():
        stats[0:1] = s1
        stats[1:2] = s2

    @pl.when(i > 0)
    def _():
        stats[0:1] = stats[0:1] + s1
        stats[1:2] = stats[1:2] + s2


def _bn_norm(stats, h2, g_ref, be_ref):
    mu = stats[0:1] / N
    var = stats[1:2] / N - mu * mu
    rstd = lax.rsqrt(var + 1e-5)
    return jnp.maximum((h2 - mu) * rstd * g_ref[...] + be_ref[...], 0.0)


def _tc_layer1_body(p_ref, w1_ref, b1_ref, w2_ref, b2_ref, g_ref,
                    be_ref, out_ref, h_buf, stats):
    ph = pl.program_id(0)
    i = pl.program_id(1)

    @pl.when(ph == 0)
    def _():
        h2 = _gin_mlp(p_ref, w1_ref, b1_ref, w2_ref, b2_ref)
        h_buf[i] = h2
        _bn_stats(stats, h2, i)

    @pl.when(ph == 1)
    def _():
        out_ref[...] = _bn_norm(stats, h_buf[i], g_ref, be_ref)


def _tc_layer2_body(p_ref, w1_ref, b1_ref, w2_ref, b2_ref, g_ref,
                    be_ref, bat_ref, wp_ref, bp_ref, out_ref, h_buf, stats,
                    pool):
    ph = pl.program_id(0)
    i = pl.program_id(1)

    @pl.when(ph == 0)
    def _():
        h2 = _gin_mlp(p_ref, w1_ref, b1_ref, w2_ref, b2_ref)
        h_buf[i] = h2
        _bn_stats(stats, h2, i)

    @pl.when(ph == 1)
    def _():
        hn = _bn_norm(stats, h_buf[i], g_ref, be_ref)
        seg = lax.broadcasted_iota(jnp.int32, (B, R), 0)
        maskf = (bat_ref[0] == seg).astype(jnp.float32)
        part = lax.dot_general(maskf, hn, (((1,), (0,)), ((), ())),
                               preferred_element_type=jnp.float32)

        @pl.when(i == 0)
        def _():
            pool[...] = part

        @pl.when(i > 0)
        def _():
            pool[...] = pool[...] + part

        @pl.when(i == NB - 1)
        def _():
            out_ref[...] = (
                jnp.dot(pool[...], wp_ref[...],
                        preferred_element_type=jnp.float32) + bp_ref[...])


def _row_map(ph, i):
    return (i * (1 - ph), 0)


def _p_map(ph, i):
    return (0, i * (1 - ph), 0)


def _const2(ph, i):
    return (0, 0)


_W_SPEC = pl.BlockSpec((D, D), _const2)
_V_SPEC = pl.BlockSpec((1, D), _const2)

_tc_layer1 = pl.pallas_call(
    _tc_layer1_body,
    grid=(2, NB),
    in_specs=[
        pl.BlockSpec((NC, R, D), _p_map),
        _W_SPEC, _V_SPEC, _W_SPEC, _V_SPEC, _V_SPEC, _V_SPEC,
    ],
    out_specs=pl.BlockSpec((R, D), lambda ph, i: (i, 0)),
    out_shape=jax.ShapeDtypeStruct((N, D), jnp.float32),
    scratch_shapes=[
        pltpu.VMEM((NB, R, D), jnp.float32),
        pltpu.VMEM((8, D), jnp.float32),
    ],
)

_tc_layer2 = pl.pallas_call(
    _tc_layer2_body,
    grid=(2, NB),
    in_specs=[
        pl.BlockSpec((NC, R, D), _p_map),
        _W_SPEC, _V_SPEC, _W_SPEC, _V_SPEC, _V_SPEC, _V_SPEC,
        pl.BlockSpec((1, 1, R), lambda ph, i: (i * ph, 0, 0)),
        _W_SPEC, _V_SPEC,
    ],
    out_specs=pl.BlockSpec((B, D), _const2),
    out_shape=jax.ShapeDtypeStruct((B, D), jnp.float32),
    scratch_shapes=[
        pltpu.VMEM((NB, R, D), jnp.float32),
        pltpu.VMEM((8, D), jnp.float32),
        pltpu.VMEM((B, D), jnp.float32),
    ],
)


def kernel(x, edge_index, degrees, identifiers, batch,
           W1_0, b1_0, W2_0, b2_0, gamma_0, beta_0,
           W1_1, b1_1, W2_1, b2_1, gamma_1, beta_1, Wp, bp):
    src = edge_index[0].astype(jnp.int32)
    dst = edge_index[1].astype(jnp.int32)
    zeros = jnp.zeros((NP, D), jnp.float32)
    bat3 = batch.astype(jnp.int32).reshape(NB, 1, R)
    wp_pad = jnp.zeros((D, D), jnp.float32).at[:, :O].set(Wp)
    bp_pad = jnp.zeros((1, D), jnp.float32).at[0, :O].set(bp)

    p1 = _sc_agg(src, dst, x, zeros)
    h1 = _tc_layer1(p1, W1_0, b1_0.reshape(1, D), W2_0,
                    b2_0.reshape(1, D), gamma_0.reshape(1, D),
                    beta_0.reshape(1, D))
    p2 = _sc_agg(src, dst, h1, zeros)
    pred = _tc_layer2(p2, W1_1, b1_1.reshape(1, D), W2_1,
                      b2_1.reshape(1, D), gamma_1.reshape(1, D),
                      beta_1.reshape(1, D), bat3, wp_pad, bp_pad)
    return pred[:, :O]


# speedup vs baseline: 1.0245x; 1.0245x over previous
import functools

import jax
import jax.numpy as jnp
from jax import lax
from jax.experimental import pallas as pl
from jax.experimental.pallas import tpu as pltpu
from jax.experimental.pallas import tpu_sc as plsc

N, D, E, B, O = 10000, 128, 320000, 16, 10

NC, NS = 2, 16
NW = NC * NS
EPW = E // NW
CHUNK = 64
NCHUNK = EPW // CHUNK
TAIL = EPW - NCHUNK * CHUNK
NP = 10240
RPT = NP // NS

R = 1000
NB = N // R


NR = 4
NG = NCHUNK // NR


def _sc_agg_body(src_hbm, dst_hbm, feat_hbm, zeros_hbm, out_hbm,
                 sidx_f, didx_r, didx_t, rows_v, rows_t, acc_sh,
                 gsem0, gsem1, gsem2, gsem3, dsem0, dsem1, dsem2, dsem3):
    gsems = (gsem0, gsem1, gsem2, gsem3)
    dsems = (dsem0, dsem1, dsem2, dsem3)
    c = lax.axis_index("c")
    s = lax.axis_index("s")
    wid = c * NS + s
    ebase = wid * EPW
    pltpu.sync_copy(src_hbm.at[pl.ds(ebase, EPW)], sidx_f)

    def gather(chunk, slot):
        return pltpu.make_async_copy(
            feat_hbm.at[sidx_f.at[pl.ds(chunk * CHUNK, CHUNK)]],
            rows_v.at[slot], gsems[slot])

    def dfetch(chunk, slot):
        return pltpu.make_async_copy(
            dst_hbm.at[pl.ds(ebase + chunk * CHUNK, CHUNK)],
            didx_r.at[slot], dsems[slot])

    for b in range(NR):
        dfetch(b, b).start()
        gather(b, b).start()

    @pl.when(c == 0)
    def _():
        @pl.when(s < NS - 1)
        def _():
            pltpu.sync_copy(feat_hbm.at[pl.ds(s * RPT, RPT)],
                            acc_sh.at[pl.ds(s * RPT, RPT)])

        @pl.when(s == NS - 1)
        def _():
            lo = (NS - 1) * RPT
            pltpu.sync_copy(feat_hbm.at[pl.ds(lo, N - lo)],
                            acc_sh.at[pl.ds(lo, N - lo)])

    @pl.when(c == 1)
    def _():
        pltpu.sync_copy(zeros_hbm.at[pl.ds(s * RPT, RPT)],
                        acc_sh.at[pl.ds(s * RPT, RPT)])
    plsc.subcore_barrier()

    def round_(g, carry):
        for b in range(NR):
            ck = g * NR + b
            gather(ck, b).wait()
            dfetch(ck, b).wait()
            pltpu.sync_copy(rows_v.at[b], acc_sh.at[didx_r.at[b]], add=True)

            @pl.when(g < NG - 1)
            def _():
                dfetch(ck + NR, b).start()
                gather(ck + NR, b).start()
        return carry

    lax.fori_loop(0, NG, round_, 0)
    pltpu.sync_copy(dst_hbm.at[pl.ds(ebase + NCHUNK * CHUNK, TAIL)], didx_t)
    pltpu.async_copy(
        feat_hbm.at[sidx_f.at[pl.ds(NCHUNK * CHUNK, TAIL)]], rows_t,
        gsem0).wait()
    pltpu.sync_copy(rows_t, acc_sh.at[didx_t], add=True)
    plsc.subcore_barrier()
    pltpu.sync_copy(acc_sh.at[pl.ds(s * RPT, RPT)],
                    out_hbm.at[c, pl.ds(s * RPT, RPT)])


_sc_agg = functools.partial(
    pl.kernel,
    out_type=jax.ShapeDtypeStruct((NC, NP, D), jnp.float32),
    mesh=plsc.VectorSubcoreMesh(core_axis_name="c", subcore_axis_name="s"),
    scratch_types=[
        pltpu.VMEM((EPW,), jnp.int32),
        pltpu.VMEM((NR, CHUNK), jnp.int32),
        pltpu.VMEM((TAIL,), jnp.int32),
        pltpu.VMEM((NR, CHUNK, D), jnp.float32),
        pltpu.VMEM((TAIL, D), jnp.float32),
        pltpu.VMEM_SHARED((NP, D), jnp.float32),
        pltpu.SemaphoreType.DMA,
        pltpu.SemaphoreType.DMA,
        pltpu.SemaphoreType.DMA,
        pltpu.SemaphoreType.DMA,
        pltpu.SemaphoreType.DMA,
        pltpu.SemaphoreType.DMA,
        pltpu.SemaphoreType.DMA,
        pltpu.SemaphoreType.DMA,
    ],
)(_sc_agg_body)


def _gin_mlp(p_ref, w1_ref, b1_ref, w2_ref, b2_ref):
    h = p_ref[0] + p_ref[1]
    a = jnp.maximum(
        jnp.dot(h, w1_ref[...], preferred_element_type=jnp.float32)
        + b1_ref[...], 0.0)
    return (jnp.dot(a, w2_ref[...], preferred_element_type=jnp.float32)
            + b2_ref[...])


def _bn_stats(stats, h2, i):
    s1 = jnp.sum(h2, axis=0, keepdims=True)
    s2 = jnp.sum(h2 * h2, axis=0, keepdims=True)

    @pl.when(i == 0)
    def _():
        stats[0:1] = s1
        stats[1:2] = s2

    @pl.when(i > 0)
    def _():
        stats[0:1] = stats[0:1] + s1
        stats[1:2] = stats[1:2] + s2


def _bn_norm(stats, h2, g_ref, be_ref):
    mu = stats[0:1] / N
    var = stats[1:2] / N - mu * mu
    rstd = lax.rsqrt(var + 1e-5)
    return jnp.maximum((h2 - mu) * rstd * g_ref[...] + be_ref[...], 0.0)


def _tc_layer1_body(p_ref, w1_ref, b1_ref, w2_ref, b2_ref, g_ref,
                    be_ref, out_ref, h_buf, stats):
    ph = pl.program_id(0)
    i = pl.program_id(1)

    @pl.when(ph == 0)
    def _():
        h2 = _gin_mlp(p_ref, w1_ref, b1_ref, w2_ref, b2_ref)
        h_buf[i] = h2
        _bn_stats(stats, h2, i)

    @pl.when(ph == 1)
    def _():
        out_ref[...] = _bn_norm(stats, h_buf[i], g_ref, be_ref)


def _tc_layer2_body(p_ref, w1_ref, b1_ref, w2_ref, b2_ref, g_ref,
                    be_ref, bat_ref, wp_ref, bp_ref, out_ref, h_buf, stats,
                    pool):
    ph = pl.program_id(0)
    i = pl.program_id(1)

    @pl.when(ph == 0)
    def _():
        h2 = _gin_mlp(p_ref, w1_ref, b1_ref, w2_ref, b2_ref)
        h_buf[i] = h2
        _bn_stats(stats, h2, i)

    @pl.when(ph == 1)
    def _():
        hn = _bn_norm(stats, h_buf[i], g_ref, be_ref)
        seg = lax.broadcasted_iota(jnp.int32, (B, R), 0)
        maskf = (bat_ref[0] == seg).astype(jnp.float32)
        part = lax.dot_general(maskf, hn, (((1,), (0,)), ((), ())),
                               preferred_element_type=jnp.float32)

        @pl.when(i == 0)
        def _():
            pool[...] = part

        @pl.when(i > 0)
        def _():
            pool[...] = pool[...] + part

        @pl.when(i == NB - 1)
        def _():
            out_ref[...] = (
                jnp.dot(pool[...], wp_ref[...],
                        preferred_element_type=jnp.float32) + bp_ref[...])


def _row_map(ph, i):
    return (i * (1 - ph), 0)


def _p_map(ph, i):
    return (0, i * (1 - ph), 0)


def _const2(ph, i):
    return (0, 0)


_W_SPEC = pl.BlockSpec((D, D), _const2)
_V_SPEC = pl.BlockSpec((1, D), _const2)

_tc_layer1 = pl.pallas_call(
    _tc_layer1_body,
    grid=(2, NB),
    in_specs=[
        pl.BlockSpec((NC, R, D), _p_map),
        _W_SPEC, _V_SPEC, _W_SPEC, _V_SPEC, _V_SPEC, _V_SPEC,
    ],
    out_specs=pl.BlockSpec((R, D), lambda ph, i: (i, 0)),
    out_shape=jax.ShapeDtypeStruct((N, D), jnp.float32),
    scratch_shapes=[
        pltpu.VMEM((NB, R, D), jnp.float32),
        pltpu.VMEM((8, D), jnp.float32),
    ],
)

_tc_layer2 = pl.pallas_call(
    _tc_layer2_body,
    grid=(2, NB),
    in_specs=[
        pl.BlockSpec((NC, R, D), _p_map),
        _W_SPEC, _V_SPEC, _W_SPEC, _V_SPEC, _V_SPEC, _V_SPEC,
        pl.BlockSpec((1, 1, R), lambda ph, i: (i * ph, 0, 0)),
        _W_SPEC, _V_SPEC,
    ],
    out_specs=pl.BlockSpec((B, D), _const2),
    out_shape=jax.ShapeDtypeStruct((B, D), jnp.float32),
    scratch_shapes=[
        pltpu.VMEM((NB, R, D), jnp.float32),
        pltpu.VMEM((8, D), jnp.float32),
        pltpu.VMEM((B, D), jnp.float32),
    ],
)


def kernel(x, edge_index, degrees, identifiers, batch,
           W1_0, b1_0, W2_0, b2_0, gamma_0, beta_0,
           W1_1, b1_1, W2_1, b2_1, gamma_1, beta_1, Wp, bp):
    src = edge_index[0].astype(jnp.int32)
    dst = edge_index[1].astype(jnp.int32)
    zeros = jnp.zeros((NP, D), jnp.float32)
    bat3 = batch.astype(jnp.int32).reshape(NB, 1, R)
    wp_pad = jnp.zeros((D, D), jnp.float32).at[:, :O].set(Wp)
    bp_pad = jnp.zeros((1, D), jnp.float32).at[0, :O].set(bp)

    p1 = _sc_agg(src, dst, x, zeros)
    h1 = _tc_layer1(p1, W1_0, b1_0.reshape(1, D), W2_0,
                    b2_0.reshape(1, D), gamma_0.reshape(1, D),
                    beta_0.reshape(1, D))
    p2 = _sc_agg(src, dst, h1, zeros)
    pred = _tc_layer2(p2, W1_1, b1_1.reshape(1, D), W2_1,
                      b2_1.reshape(1, D), gamma_1.reshape(1, D),
                      beta_1.reshape(1, D), bat3, wp_pad, bp_pad)
    return pred[:, :O]
